# baseline (speedup 1.0000x reference)
import jax
import jax.numpy as jnp
from jax import lax
from jax.experimental import pallas as pl
from jax.experimental.pallas import tpu as pltpu
from jax.experimental.pallas import tpu_sc as plsc

N_ATOMS = 10000
N_PAIRS = 320000
C = 128
H = C // 2
NSUB = 16
EDGES_PER_SUB = N_PAIRS // NSUB
K = 16
CPB = 5
BK = K * CPB
NBLK = EDGES_PER_SUB // BK
NCHUNK = EDGES_PER_SUB // K
ROWS_PER_SUB = N_ATOMS // NSUB
WB = 5
NWB = ROWS_PER_SUB // WB


def _sc_call(i2d, j2d, p3h0, p3h1, i1r, d3):
    mesh = plsc.VectorSubcoreMesh(core_axis_name="c", subcore_axis_name="s")

    def body(i2d_hbm, j2d_hbm, p3h0_hbm, p3h1_hbm, i1r_hbm, d3_hbm,
             ph_out, dot_out,
             acc,
             rows0, rows1, i1h2,
             iblk0, iblk1, jblk0, jblk1, d3blk,
             siidx0, siidx1,
             sixb0, sixb1, sdat0, sdat1, ssc0, ssc1, si1):
        c = lax.axis_index("c")
        s = lax.axis_index("s")
        rows = (rows0, rows1)
        iblk = (iblk0, iblk1)
        jblk = (jblk0, jblk1)
        siidx = (siidx0, siidx1)
        sixb = (sixb0, sixb1)
        sdat = (sdat0, sdat1)
        ssc = (ssc0, ssc1)

        def issue_blk(B, bp):
            row = s * NBLK + B
            pltpu.async_copy(i2d_hbm.at[row], iblk[bp], sixb[bp])
            pltpu.async_copy(j2d_hbm.at[row], jblk[bp], sixb[bp])

        def wait_blk(bp):
            pltpu.make_async_copy(i2d_hbm.at[0], iblk[bp], sixb[bp]).wait()
            pltpu.make_async_copy(j2d_hbm.at[0], jblk[bp], sixb[bp]).wait()

        def issue_data(n, p, bp, u):
            e0 = (s * NCHUNK + n) * K
            jslice = jblk[bp].at[u]

            @pl.when(c == 0)
            def _():
                pltpu.async_copy(p3h0_hbm.at[jslice], rows[p], sdat[p])

            @pl.when(c == 1)
            def _():
                pltpu.async_copy(p3h1_hbm.at[jslice], rows[p], sdat[p])


        def wait_data(p):
            @pl.when(c == 0)
            def _():
                pltpu.make_async_copy(
                    p3h0_hbm.at[jblk0.at[0]], rows[p],
                    sdat[p]).wait()

            @pl.when(c == 1)
            def _():
                pltpu.make_async_copy(
                    p3h1_hbm.at[jblk0.at[0]], rows[p],
                    sdat[p]).wait()


        def issue_i1_pair(n_even):
            e0 = (s * NCHUNK + n_even) * K
            pltpu.async_copy(i1r_hbm.at[pl.ds(e0, 2 * K), c], i1h2, si1)

        def wait_i1_pair():
            pltpu.make_async_copy(
                i1r_hbm.at[pl.ds(0, 2 * K), c], i1h2, si1).wait()

        def wait_scatter(p):
            pltpu.make_async_copy(rows[p], acc.at[siidx[p]], ssc[p]).wait()

        def compute(p, u):
            rv = rows[p]

            def edge(k, ecarry):
                ws = [i1h2[p * K + k, pl.ds(b * 16, 16)] for b in range(4)]
                kk = jnp.full((16,), u * K + k, jnp.int32)
                for x in range(3):
                    dsc = plsc.load_gather(
                        d3blk, [kk, jnp.full((16,), x, jnp.int32)])
                    for b in range(4):
                        r = rv[k, x, pl.ds(b * 16, 16)]
                        rv[k, x, pl.ds(b * 16, 16)] = ws[b] * (r + dsc)
                return ecarry
            lax.fori_loop(0, K, edge, 0)

        def sub(B, u, bp):
            n = B * CPB + u
            p = (u + bp) & 1
            q = 1 - p

            if u == 0:
                @pl.when(n >= 1)
                def _():
                    wait_scatter(q)
            else:
                wait_scatter(q)

            if u < CPB - 1:
                issue_data(n + 1, q, bp, u + 1)
            else:
                @pl.when(n < NCHUNK - 1)
                def _():
                    wait_blk(1 - bp)
                    issue_data(n + 1, q, 1 - bp, 0)

            wait_data(p)
            if p == 0:
                wait_i1_pair()
            compute(p, u)
            if p == 1:
                @pl.when(n < NCHUNK - 1)
                def _():
                    issue_i1_pair(n + 1)
            siidx[p][pl.ds(0, K)] = iblk[bp][u, pl.ds(0, K)]
            pltpu.async_copy(rows[p], acc.at[siidx[p]], ssc[p], add=True)

        def block(B, bp):
            pltpu.sync_copy(d3_hbm.at[pl.ds((s * NBLK + B) * BK, BK)], d3blk)
            for u in range(CPB):
                sub(B, u, bp)

            @pl.when(B + 2 <= NBLK - 1)
            def _():
                issue_blk(B + 2, bp)

        def zrow(k, carry):
            for x in range(3):
                for b in range(4):
                    rows0[k, x, pl.ds(b * 16, 16)] = jnp.zeros((16,), jnp.float32)
            return carry
        lax.fori_loop(0, K, zrow, 0)

        def zblk(t, carry):
            pltpu.sync_copy(rows0, acc.at[pl.ds(s * ROWS_PER_SUB + t * K, K)])
            return carry
        lax.fori_loop(0, ROWS_PER_SUB // K, zblk, 0)
        pltpu.sync_copy(rows0.at[pl.ds(0, 1)],
                        acc.at[pl.ds(s * ROWS_PER_SUB + 624, 1)])
        plsc.subcore_barrier()

        issue_blk(0, 0)
        issue_blk(1, 1)
        wait_blk(0)
        issue_data(0, 0, 0, 0)
        issue_i1_pair(0)

        def pipe(G, carry):
            block(2 * G, 0)
            block(2 * G + 1, 1)
            return carry
        lax.fori_loop(0, NBLK // 2, pipe, 0)
        wait_scatter(1)
        plsc.subcore_barrier()

        def wblk(t, carry):
            r0 = s * ROWS_PER_SUB + t * WB
            pltpu.sync_copy(acc.at[pl.ds(r0, WB)], rows0.at[pl.ds(0, WB)])
            pltpu.sync_copy(rows0.at[pl.ds(0, WB)], ph_out.at[c, pl.ds(r0, WB)])

            def drow(k, kcarry):
                for b in range(4):
                    a0 = rows0[k, 0, pl.ds(b * 16, 16)]
                    a1 = rows0[k, 1, pl.ds(b * 16, 16)]
                    a2 = rows0[k, 2, pl.ds(b * 16, 16)]
                    i1h2[k, pl.ds(b * 16, 16)] = a0 * a0 + a1 * a1 + a2 * a2
                return kcarry
            lax.fori_loop(0, WB, drow, 0)
            pltpu.sync_copy(i1h2.at[pl.ds(0, WB)], dot_out.at[c, pl.ds(r0, WB)])
            return carry
        lax.fori_loop(0, NWB, wblk, 0)

    fn = pl.kernel(
        body,
        out_type=(
            jax.ShapeDtypeStruct((2, N_ATOMS, 3, H), jnp.float32),
            jax.ShapeDtypeStruct((2, N_ATOMS, H), jnp.float32),
        ),
        mesh=mesh,
        compiler_params=pltpu.CompilerParams(
            use_tc_tiling_on_sc=False, needs_layout_passes=False),
        scratch_types=(
            pltpu.VMEM_SHARED((N_ATOMS, 3, H), jnp.float32),
            pltpu.VMEM((K, 3, H), jnp.float32),
            pltpu.VMEM((K, 3, H), jnp.float32),
            pltpu.VMEM((2 * K, H), jnp.float32),
            pltpu.VMEM((CPB, K), jnp.int32),
            pltpu.VMEM((CPB, K), jnp.int32),
            pltpu.VMEM((CPB, K), jnp.int32),
            pltpu.VMEM((CPB, K), jnp.int32),
            pltpu.VMEM((BK, 3), jnp.float32),
            pltpu.VMEM((K,), jnp.int32),
            pltpu.VMEM((K,), jnp.int32),
            pltpu.SemaphoreType.DMA,
            pltpu.SemaphoreType.DMA,
            pltpu.SemaphoreType.DMA,
            pltpu.SemaphoreType.DMA,
            pltpu.SemaphoreType.DMA,
            pltpu.SemaphoreType.DMA,
            pltpu.SemaphoreType.DMA,
        ),
    )
    return fn(i2d, j2d, p3h0, p3h1, i1r, d3)


def kernel(ind_2, p3, i1, d3):
    i2d = ind_2[:, 0].reshape(N_PAIRS // BK, CPB, K)
    j2d = ind_2[:, 1].reshape(N_PAIRS // BK, CPB, K)
    p3h0 = p3[:, :, :H]
    p3h1 = p3[:, :, H:]
    i1r = i1.reshape(N_PAIRS, 2, H)
    ph, dt = _sc_call(i2d, j2d, p3h0, p3h1, i1r, d3)
    p3_new = jnp.concatenate([ph[0], ph[1]], axis=-1)
    dotted = jnp.concatenate([dt[0], dt[1]], axis=-1)
    return (p3_new, dotted)


# speedup vs baseline: 1.0549x; 1.0549x over previous
import jax
import jax.numpy as jnp
from jax import lax
from jax.experimental import pallas as pl
from jax.experimental.pallas import tpu as pltpu
from jax.experimental.pallas import tpu_sc as plsc

N_ATOMS = 10000
N_PAIRS = 320000
C = 128
H = C // 2
NSUB = 16
EDGES_PER_SUB = N_PAIRS // NSUB
K = 16
CPB = 5
BK = K * CPB
NBLK = EDGES_PER_SUB // BK
NCHUNK = EDGES_PER_SUB // K
ROWS_PER_SUB = N_ATOMS // NSUB
WB = 5
NWB = ROWS_PER_SUB // WB


def _sc_call(i2d, j2d, p3h0, p3h1, i1r, d3):
    mesh = plsc.VectorSubcoreMesh(core_axis_name="c", subcore_axis_name="s")

    def body(i2d_hbm, j2d_hbm, p3h0_hbm, p3h1_hbm, i1r_hbm, d3_hbm,
             ph_out, dot_out,
             acc,
             rows0, rows1, i1h0, i1h1,
             iblk0, iblk1, jblk0, jblk1, d3blk,
             siidx0, siidx1,
             sixb0, sixb1, sdat0, sdat1, ssc0, ssc1):
        c = lax.axis_index("c")
        s = lax.axis_index("s")
        rows = (rows0, rows1)
        i1h = (i1h0, i1h1)
        iblk = (iblk0, iblk1)
        jblk = (jblk0, jblk1)
        siidx = (siidx0, siidx1)
        sixb = (sixb0, sixb1)
        sdat = (sdat0, sdat1)
        ssc = (ssc0, ssc1)

        def issue_blk(B, bp):
            row = s * NBLK + B
            pltpu.async_copy(i2d_hbm.at[row], iblk[bp], sixb[bp])
            pltpu.async_copy(j2d_hbm.at[row], jblk[bp], sixb[bp])

        def wait_blk(bp):
            pltpu.make_async_copy(i2d_hbm.at[0], iblk[bp], sixb[bp]).wait()
            pltpu.make_async_copy(j2d_hbm.at[0], jblk[bp], sixb[bp]).wait()

        def issue_data(n, p, bp, u):
            e0 = (s * NCHUNK + n) * K
            jslice = jblk[bp].at[u]

            @pl.when(c == 0)
            def _():
                pltpu.async_copy(p3h0_hbm.at[jslice], rows[p], sdat[p])

            @pl.when(c == 1)
            def _():
                pltpu.async_copy(p3h1_hbm.at[jslice], rows[p], sdat[p])

            pltpu.async_copy(i1r_hbm.at[pl.ds(e0, K), c], i1h[p], sdat[p])

        def wait_data(p):
            @pl.when(c == 0)
            def _():
                pltpu.make_async_copy(
                    p3h0_hbm.at[jblk0.at[0]], rows[p],
                    sdat[p]).wait()

            @pl.when(c == 1)
            def _():
                pltpu.make_async_copy(
                    p3h1_hbm.at[jblk0.at[0]], rows[p],
                    sdat[p]).wait()

            pltpu.make_async_copy(
                i1r_hbm.at[pl.ds(0, K), c], i1h[p], sdat[p]).wait()

        def wait_scatter(p):
            pltpu.make_async_copy(rows[p], acc.at[siidx[p]], ssc[p]).wait()

        def compute(p, u):
            rv = rows[p]
            iv = i1h[p]

            def edge(k, ecarry):
                ws = [iv[k, pl.ds(b * 16, 16)] for b in range(4)]
                kk = jnp.full((16,), u * K + k, jnp.int32)
                for x in range(3):
                    dsc = plsc.load_gather(
                        d3blk, [kk, jnp.full((16,), x, jnp.int32)])
                    for b in range(4):
                        r = rv[k, x, pl.ds(b * 16, 16)]
                        rv[k, x, pl.ds(b * 16, 16)] = ws[b] * (r + dsc)
                return ecarry
            lax.fori_loop(0, K, edge, 0)

        def sub(B, u, bp):
            n = B * CPB + u
            p = (u + bp) & 1
            q = 1 - p

            if u == 0:
                @pl.when(n >= 1)
                def _():
                    wait_scatter(q)
            else:
                wait_scatter(q)

            if u < CPB - 1:
                issue_data(n + 1, q, bp, u + 1)
            else:
                @pl.when(n < NCHUNK - 1)
                def _():
                    wait_blk(1 - bp)
                    issue_data(n + 1, q, 1 - bp, 0)

            wait_data(p)
            compute(p, u)
            siidx[p][pl.ds(0, K)] = iblk[bp][u, pl.ds(0, K)]
            pltpu.async_copy(rows[p], acc.at[siidx[p]], ssc[p], add=True)

        def block(B, bp):
            pltpu.sync_copy(d3_hbm.at[pl.ds((s * NBLK + B) * BK, BK)], d3blk)
            for u in range(CPB):
                sub(B, u, bp)

            @pl.when(B + 2 <= NBLK - 1)
            def _():
                issue_blk(B + 2, bp)

        def zrow(k, carry):
            for x in range(3):
                for b in range(4):
                    rows0[k, x, pl.ds(b * 16, 16)] = jnp.zeros((16,), jnp.float32)
            return carry
        lax.fori_loop(0, K, zrow, 0)

        def zblk(t, carry):
            pltpu.sync_copy(rows0, acc.at[pl.ds(s * ROWS_PER_SUB + t * K, K)])
            return carry
        lax.fori_loop(0, ROWS_PER_SUB // K, zblk, 0)
        pltpu.sync_copy(rows0.at[pl.ds(0, 1)],
                        acc.at[pl.ds(s * ROWS_PER_SUB + 624, 1)])
        plsc.subcore_barrier()

        issue_blk(0, 0)
        issue_blk(1, 1)
        wait_blk(0)
        issue_data(0, 0, 0, 0)

        def pipe(G, carry):
            block(2 * G, 0)
            block(2 * G + 1, 1)
            return carry
        lax.fori_loop(0, NBLK // 2, pipe, 0)
        wait_scatter(1)
        plsc.subcore_barrier()

        def wblk(t, carry):
            r0 = s * ROWS_PER_SUB + t * WB
            pltpu.sync_copy(acc.at[pl.ds(r0, WB)], rows0.at[pl.ds(0, WB)])
            pltpu.sync_copy(rows0.at[pl.ds(0, WB)], ph_out.at[c, pl.ds(r0, WB)])

            def drow(k, kcarry):
                for b in range(4):
                    a0 = rows0[k, 0, pl.ds(b * 16, 16)]
                    a1 = rows0[k, 1, pl.ds(b * 16, 16)]
                    a2 = rows0[k, 2, pl.ds(b * 16, 16)]
                    i1h0[k, pl.ds(b * 16, 16)] = a0 * a0 + a1 * a1 + a2 * a2
                return kcarry
            lax.fori_loop(0, WB, drow, 0)
            pltpu.sync_copy(i1h0.at[pl.ds(0, WB)], dot_out.at[c, pl.ds(r0, WB)])
            return carry
        lax.fori_loop(0, NWB, wblk, 0)

    fn = pl.kernel(
        body,
        out_type=(
            jax.ShapeDtypeStruct((2, N_ATOMS, 3, H), jnp.float32),
            jax.ShapeDtypeStruct((2, N_ATOMS, H), jnp.float32),
        ),
        mesh=mesh,
        compiler_params=pltpu.CompilerParams(
            use_tc_tiling_on_sc=False, needs_layout_passes=False),
        scratch_types=(
            pltpu.VMEM_SHARED((N_ATOMS, 3, H), jnp.float32),
            pltpu.VMEM((K, 3, H), jnp.float32),
            pltpu.VMEM((K, 3, H), jnp.float32),
            pltpu.VMEM((K, H), jnp.float32),
            pltpu.VMEM((K, H), jnp.float32),
            pltpu.VMEM((CPB, K), jnp.int32),
            pltpu.VMEM((CPB, K), jnp.int32),
            pltpu.VMEM((CPB, K), jnp.int32),
            pltpu.VMEM((CPB, K), jnp.int32),
            pltpu.VMEM((BK, 3), jnp.float32),
            pltpu.VMEM((K,), jnp.int32),
            pltpu.VMEM((K,), jnp.int32),
            pltpu.SemaphoreType.DMA,
            pltpu.SemaphoreType.DMA,
            pltpu.SemaphoreType.DMA,
            pltpu.SemaphoreType.DMA,
            pltpu.SemaphoreType.DMA,
            pltpu.SemaphoreType.DMA,
        ),
    )
    return fn(i2d, j2d, p3h0, p3h1, i1r, d3)


def kernel(ind_2, p3, i1, d3):
    i2d = ind_2[:, 0].reshape(N_PAIRS // BK, CPB, K)
    j2d = ind_2[:, 1].reshape(N_PAIRS // BK, CPB, K)
    p3h0 = p3[:, :, :H]
    p3h1 = p3[:, :, H:]
    i1r = i1.reshape(N_PAIRS, 2, H)
    ph, dt = _sc_call(i2d, j2d, p3h0, p3h1, i1r, d3)
    p3_new = jnp.concatenate([ph[0], ph[1]], axis=-1)
    dotted = jnp.concatenate([dt[0], dt[1]], axis=-1)
    return (p3_new, dotted)
